# Initial kernel scaffold; baseline (speedup 1.0000x reference)
import jax
import jax.numpy as jnp
from jax.experimental import pallas as pl


def kernel(x, gate_W, gate_b, expert_W, expert_b):
    raise NotImplementedError("write your pallas kernel here")


# speedup vs baseline: 5.4149x; 5.4149x over previous
import functools

import jax
import jax.numpy as jnp
from jax.experimental import pallas as pl
from jax.experimental.pallas import tpu as pltpu

IN_DIM = 768
NUM_EXPERTS = 8
TOP_K = 2
BLK = 512


def _moe_block(x_ref, gw_ref, gb_ref, ew_ref, eb_ref, out_ref):
    x = x_ref[...]
    logits = jax.lax.dot_general(
        x, gw_ref[...], (((1,), (0,)), ((), ())),
        preferred_element_type=jnp.float32) + gb_ref[...]

    iota = jax.lax.broadcasted_iota(jnp.int32, logits.shape, 1)
    m1 = jnp.max(logits, axis=1, keepdims=True)
    i1 = jnp.min(jnp.where(logits == m1, iota, NUM_EXPERTS), axis=1,
                 keepdims=True)
    oh1 = (iota == i1)
    masked = jnp.where(oh1, -jnp.inf, logits)
    m2 = jnp.max(masked, axis=1, keepdims=True)
    i2 = jnp.min(jnp.where(masked == m2, iota, NUM_EXPERTS), axis=1,
                 keepdims=True)
    oh2 = (iota == i2)
    w1 = 1.0 / (1.0 + jnp.exp(m2 - m1))
    w2 = 1.0 - w1
    wdense = jnp.where(oh1, w1, 0.0) + jnp.where(oh2, w2, 0.0)

    acc = jnp.zeros((x.shape[0], IN_DIM), jnp.float32)
    for e in range(NUM_EXPERTS):
        h = jax.lax.dot_general(
            x, ew_ref[e], (((1,), (0,)), ((), ())),
            preferred_element_type=jnp.float32) + eb_ref[e][None, :]
        acc = acc + jnp.maximum(h, 0.0) * wdense[:, e][:, None]
    out_ref[...] = acc


@jax.jit
def kernel(x, gate_W, gate_b, expert_W, expert_b):
    B = x.shape[0]
    grid = (B // BLK,)
    return pl.pallas_call(
        _moe_block,
        grid=grid,
        in_specs=[
            pl.BlockSpec((BLK, IN_DIM), lambda i: (i, 0)),
            pl.BlockSpec((IN_DIM, NUM_EXPERTS), lambda i: (0, 0)),
            pl.BlockSpec((NUM_EXPERTS,), lambda i: (0,)),
            pl.BlockSpec((NUM_EXPERTS, IN_DIM, IN_DIM), lambda i: (0, 0, 0)),
            pl.BlockSpec((NUM_EXPERTS, IN_DIM), lambda i: (0, 0)),
        ],
        out_specs=pl.BlockSpec((BLK, IN_DIM), lambda i: (i, 0)),
        out_shape=jax.ShapeDtypeStruct((B, IN_DIM), jnp.float32),
    )(x, gate_W, gate_b, expert_W, expert_b)
